# baseline (speedup 1.0000x reference)
import functools
import math

import jax
import jax.numpy as jnp
from jax import lax
from jax.experimental import pallas as pl
from jax.experimental.pallas import tpu as pltpu
from jax.experimental.pallas import tpu_sc as plsc

NEG_INF = float("-inf")
K = 64


def _scores_body(q_ref, k_ref, o_ref):
    q = q_ref[:]
    k = k_ref[:]
    s = lax.dot_general(q, k, (((1,), (1,)), ((), ())),
                        preferred_element_type=jnp.float32,
                        precision=lax.Precision.DEFAULT)
    o_ref[:] = s * (1.0 / math.sqrt(q.shape[1]))


def _scores(qf, keys, rb, mb):
    n, h = qf.shape
    m = keys.shape[0]
    grid = (m // mb, n // rb)
    return pl.pallas_call(
        _scores_body,
        grid=grid,
        in_specs=[
            pl.BlockSpec((rb, h), lambda j, i: (i, 0)),
            pl.BlockSpec((mb, h), lambda j, i: (j, 0)),
        ],
        out_specs=pl.BlockSpec((rb, mb), lambda j, i: (i, j)),
        out_shape=jax.ShapeDtypeStruct((n, m), jnp.float32),
    )(qf, keys)


def _topk_body(s_ref, top_ref, attn_ref):
    s = s_ref[:]
    r = s.shape[0]
    prev = jnp.full((r, 1), jnp.inf, jnp.float32)
    cnt = jnp.zeros((r, 1), jnp.int32)
    vals = []
    for _ in range(K):
        masked = jnp.where(s < prev, s, NEG_INF)
        newmax = jnp.max(masked, axis=1, keepdims=True)
        newcnt = jnp.sum((masked == newmax).astype(jnp.int32), axis=1,
                         keepdims=True)
        need_new = cnt <= 0
        val = jnp.where(need_new, newmax, prev)
        cnt = jnp.where(need_new, newcnt, cnt) - 1
        prev = val
        vals.append(val)
    top = jnp.concatenate(vals, axis=1)
    top_ref[:] = top
    mx = top[:, 0:1]
    e = jnp.exp(top - mx)
    attn_ref[:] = e / jnp.sum(e, axis=1, keepdims=True)


def _topk(scores, rb):
    n, m = scores.shape
    return pl.pallas_call(
        _topk_body,
        grid=(n // rb,),
        in_specs=[pl.BlockSpec((rb, m), lambda i: (i, 0))],
        out_specs=[
            pl.BlockSpec((rb, K), lambda i: (i, 0)),
            pl.BlockSpec((rb, K), lambda i: (i, 0)),
        ],
        out_shape=[
            jax.ShapeDtypeStruct((n, K), jnp.float32),
            jax.ShapeDtypeStruct((n, K), jnp.float32),
        ],
    )(scores)


_NROWS_PER_TEC = 32
_M = 8192


def _perm(x, idx):
    return jnp.take_along_axis(x, idx, axis=0, mode="promise_in_bounds")


def _lane_mask(bit):
    t = lax.iota(jnp.int32, 16) & bit
    return (t - 1) >> 31


def _sel(m, a, b):
    ai = lax.bitcast_convert_type(a, jnp.int32)
    bi = lax.bitcast_convert_type(b, jnp.int32)
    return lax.bitcast_convert_type((ai & m) | (bi & ~m), jnp.float32)


def _sort16_desc(x):
    iota = lax.iota(jnp.int32, 16)
    for k in (2, 4, 8, 16):
        dk = _lane_mask(k)
        j = k >> 1
        while j:
            p = _perm(x, iota ^ j)
            tm = ~(_lane_mask(j) ^ dk)
            x = _sel(tm, jnp.maximum(x, p), jnp.minimum(x, p))
            j >>= 1
    return x


def _clean16_desc(x):
    iota = lax.iota(jnp.int32, 16)
    for j in (8, 4, 2, 1):
        p = _perm(x, iota ^ j)
        x = _sel(_lane_mask(j), jnp.maximum(x, p), jnp.minimum(x, p))
    return x


def _rev(x):
    return lax.rev(x, (0,))


def _merge16(a, b):
    rb = _rev(b)
    return (_clean16_desc(jnp.maximum(a, rb)),
            _clean16_desc(jnp.minimum(a, rb)))


def _merge32(p, q):
    r0, r1 = _rev(q[1]), _rev(q[0])
    h0, h1 = jnp.maximum(p[0], r0), jnp.maximum(p[1], r1)
    l0, l1 = jnp.minimum(p[0], r0), jnp.minimum(p[1], r1)
    return (_clean16_desc(jnp.maximum(h0, h1)),
            _clean16_desc(jnp.minimum(h0, h1)),
            _clean16_desc(jnp.maximum(l0, l1)),
            _clean16_desc(jnp.minimum(l0, l1)))


def _group64(v0, v1, v2, v3):
    s = [_sort16_desc(v) for v in (v0, v1, v2, v3)]
    return _merge32(_merge16(s[0], s[1]), _merge16(s[2], s[3]))


def _merge64keep(a, b):
    r = (_rev(b[3]), _rev(b[2]), _rev(b[1]), _rev(b[0]))
    h = [jnp.maximum(a[i], r[i]) for i in range(4)]
    p0, p1 = jnp.maximum(h[0], h[2]), jnp.maximum(h[1], h[3])
    p2, p3 = jnp.minimum(h[0], h[2]), jnp.minimum(h[1], h[3])
    q = (jnp.maximum(p0, p1), jnp.minimum(p0, p1),
         jnp.maximum(p2, p3), jnp.minimum(p2, p3))
    return tuple(_clean16_desc(x) for x in q)


def _bfly(x, op):
    iota = lax.iota(jnp.int32, 16)
    for j in (1, 2, 4, 8):
        x = op(x, _perm(x, iota ^ j))
    return x


def _merge64x16(a, v_sorted):
    h3 = jnp.maximum(a[3], _rev(v_sorted))
    p1, p3 = jnp.maximum(a[1], h3), jnp.minimum(a[1], h3)
    q0, q1 = jnp.maximum(a[0], p1), jnp.minimum(a[0], p1)
    q2, q3 = jnp.maximum(a[2], p3), jnp.minimum(a[2], p3)
    return tuple(_clean16_desc(x) for x in (q0, q1, q2, q3))


def _sc_topk_body(scores_hbm, top_hbm, attn_hbm, row_a, row_b, gmax_v,
                  top_v, attn_v, sem_a, sem_b):
    wid = lax.axis_index("s") * 2 + lax.axis_index("c")
    ninf = jnp.full((16,), NEG_INF, jnp.float32)
    base = wid * _NROWS_PER_TEC
    zero16 = lax.iota(jnp.int32, 16) * 0

    def process(row, row_v):
        def gm_body(gq, _):
            comb = jnp.zeros((16,), jnp.int32)
            for u in range(16):
                g = gq * 16 + u
                v = row_v[pl.ds(g * 64, 16)]
                for k in range(1, 4):
                    v = jnp.maximum(v, row_v[pl.ds(g * 64 + k * 16, 16)])
                sp = _bfly(v, jnp.maximum)
                t = lax.iota(jnp.int32, 16) ^ u
                lane_u = (t - 1) >> 31
                comb = comb | (lax.bitcast_convert_type(sp, jnp.int32) &
                               lane_u)
            gmax_v[pl.ds(gq * 16, 16)] = lax.bitcast_convert_type(
                comb, jnp.float32)
            return 0

        lax.fori_loop(0, 8, gm_body, 0)
        gm = [gmax_v[pl.ds(k * 16, 16)] for k in range(8)]
        pre = _merge64keep(_group64(*gm[:4]), _group64(*gm[4:]))
        t0 = _bfly(pre[3], jnp.minimum)[0]
        for k in range(4):
            top_v[pl.ds(k * 16, 16)] = ninf

        def g_body(g, tfs):
            def hit(tfs_):
                vs = [row_v[pl.ds(g * 64 + k * 16, 16)] for k in range(4)]
                a = tuple(top_v[pl.ds(i * 16, 16)] for i in range(4))
                m = _merge64keep(a, _group64(*vs))
                for i in range(4):
                    top_v[pl.ds(i * 16, 16)] = m[i]
                nt = _bfly(m[3], jnp.minimum)[0]
                return jnp.maximum(tfs_, nt)

            def miss(tfs_):
                return tfs_

            gms = gmax_v[pl.ds(g, 16)][0]
            return lax.cond(gms >= tfs, hit, miss, tfs)

        lax.fori_loop(0, 128, g_body, t0)

        acc = [top_v[pl.ds(k * 16, 16)] for k in range(4)]
        mx = _bfly(acc[0], jnp.maximum)
        es = [jnp.exp(a - mx) for a in acc]
        denom = _bfly(es[0] + es[1] + es[2] + es[3], jnp.add)
        for k in range(4):
            attn_v[pl.ds(k * 16, 16)] = es[k] / denom
        pltpu.sync_copy(top_v, top_hbm.at[row])
        pltpu.sync_copy(attn_v, attn_hbm.at[row])

    pltpu.async_copy(scores_hbm.at[base], row_a, sem_a).wait()

    def pair_body(i, _):
        ra = base + i * 2
        pltpu.async_copy(scores_hbm.at[ra + 1], row_b, sem_b)
        process(ra, row_a)
        pltpu.make_async_copy(scores_hbm.at[ra + 1], row_b, sem_b).wait()

        @pl.when(i < _NROWS_PER_TEC // 2 - 1)
        def _():
            pltpu.async_copy(scores_hbm.at[ra + 2], row_a, sem_a)

        process(ra + 1, row_b)

        @pl.when(i < _NROWS_PER_TEC // 2 - 1)
        def _():
            pltpu.make_async_copy(scores_hbm.at[ra + 2], row_a, sem_a).wait()

        return 0

    lax.fori_loop(0, _NROWS_PER_TEC // 2, pair_body, 0)


def _sc_topk(scores):
    n_rows, m = scores.shape
    mesh = plsc.VectorSubcoreMesh(core_axis_name="c", subcore_axis_name="s")
    f = pl.kernel(
        _sc_topk_body,
        mesh=mesh,
        out_type=[jax.ShapeDtypeStruct((n_rows, K), jnp.float32),
                  jax.ShapeDtypeStruct((n_rows, K), jnp.float32)],
        scratch_types=[pltpu.VMEM((m,), jnp.float32),
                       pltpu.VMEM((m,), jnp.float32),
                       pltpu.VMEM((144,), jnp.float32),
                       pltpu.VMEM((K,), jnp.float32),
                       pltpu.VMEM((K,), jnp.float32),
                       pltpu.SemaphoreType.DMA,
                       pltpu.SemaphoreType.DMA],
    )
    return f(scores)


def _out_body(s_ref, top_ref, q_ref, v_ref, gw_ref, gb_ref, o_ref, w_ref):
    @pl.when(pl.program_id(1) == 0)
    def _build_w():
        s = s_ref[:]
        top = top_ref[:]
        t = top[:, K - 1:K]
        mx = top[:, 0:1]
        denom = jnp.sum(jnp.exp(top - mx), axis=1, keepdims=True)
        gt = s > t
        n_gt = jnp.sum(gt.astype(jnp.int32), axis=1, keepdims=True)
        c_allow = K - n_gt
        eq = s == t
        iota = lax.broadcasted_iota(jnp.int32, s.shape, 1)

        def bs_body(_, carry):
            lo, hi = carry
            mid = (lo + hi) >> 1
            cnt = jnp.sum((eq & (iota <= mid)).astype(jnp.int32), axis=1,
                          keepdims=True)
            pred = cnt >= c_allow
            return jnp.where(pred, lo, mid + 1), jnp.where(pred, mid, hi)

        lo0 = jnp.zeros_like(c_allow)
        hi0 = jnp.full_like(c_allow, s.shape[1] - 1)
        _, bound = lax.fori_loop(0, 13, bs_body, (lo0, hi0))
        sel = gt | (eq & (iota <= bound) & (c_allow > 0))
        e = jnp.exp(s - mx)
        w_ref[:] = jnp.where(sel, e, 0.0) / denom

    w = w_ref[:].astype(jnp.bfloat16)
    v = v_ref[:]
    out = lax.dot_general(w, v, (((1,), (0,)), ((), ())),
                          preferred_element_type=jnp.float32,
                          precision=lax.Precision.DEFAULT)
    g = jnp.sum(q_ref[:] * gw_ref[:], axis=1, keepdims=True)
    gate = jax.nn.sigmoid(g + gb_ref[0, 0])
    o_ref[:] = out * gate


def _output(scores, top, qf, values, gate_w, gate_b, rb, hb):
    n, m = scores.shape
    h = values.shape[1]
    grid = (n // rb, h // hb)
    return pl.pallas_call(
        _out_body,
        grid=grid,
        in_specs=[
            pl.BlockSpec((rb, m), lambda i, j: (i, 0)),
            pl.BlockSpec((rb, K), lambda i, j: (i, 0)),
            pl.BlockSpec((rb, qf.shape[1]), lambda i, j: (i, 0)),
            pl.BlockSpec((m, hb), lambda i, j: (0, j)),
            pl.BlockSpec((1, qf.shape[1]), lambda i, j: (0, 0)),
            pl.BlockSpec((1, 1), lambda i, j: (0, 0), memory_space=pltpu.SMEM),
        ],
        out_specs=pl.BlockSpec((rb, hb), lambda i, j: (i, j)),
        out_shape=jax.ShapeDtypeStruct((n, h), jnp.float32),
        scratch_shapes=[pltpu.VMEM((rb, m), jnp.float32)],
    )(scores, top, qf, values, gate_w, gate_b)


def kernel(query, memory_keys, memory_values, gate_w, gate_b, top_k):
    b, q, h = query.shape
    n = b * q
    qf = query.reshape(n, h)
    scores = _scores(qf.astype(jnp.bfloat16),
                     memory_keys.astype(jnp.bfloat16), rb=128, mb=1024)
    top, attn = _sc_topk(scores)
    gb = jnp.reshape(gate_b, (1, 1)).astype(jnp.float32)
    out = _output(scores, top, qf, memory_values.astype(jnp.bfloat16),
                  gate_w, gb, rb=256, hb=256)
    return out.reshape(b, q, h), attn.reshape(b, q, K)


# speedup vs baseline: 1.2111x; 1.2111x over previous
import functools
import math

import jax
import jax.numpy as jnp
from jax import lax
from jax.experimental import pallas as pl
from jax.experimental.pallas import tpu as pltpu
from jax.experimental.pallas import tpu_sc as plsc

NEG_INF = float("-inf")
K = 64


def _scores_body(q_ref, k_ref, o_ref):
    q = q_ref[:]
    k = k_ref[:]
    s = lax.dot_general(q, k, (((1,), (1,)), ((), ())),
                        preferred_element_type=jnp.float32,
                        precision=lax.Precision.DEFAULT)
    o_ref[:] = s * (1.0 / math.sqrt(q.shape[1]))


def _scores(qf, keys, rb, mb):
    n, h = qf.shape
    m = keys.shape[0]
    del rb
    grid = (m // mb,)
    return pl.pallas_call(
        _scores_body,
        grid=grid,
        in_specs=[
            pl.BlockSpec((n, h), lambda j: (0, 0)),
            pl.BlockSpec((mb, h), lambda j: (j, 0)),
        ],
        out_specs=pl.BlockSpec((n, mb), lambda j: (0, j)),
        out_shape=jax.ShapeDtypeStruct((n, m), jnp.float32),
    )(qf, keys)


def _topk_body(s_ref, top_ref, attn_ref):
    s = s_ref[:]
    r = s.shape[0]
    prev = jnp.full((r, 1), jnp.inf, jnp.float32)
    cnt = jnp.zeros((r, 1), jnp.int32)
    vals = []
    for _ in range(K):
        masked = jnp.where(s < prev, s, NEG_INF)
        newmax = jnp.max(masked, axis=1, keepdims=True)
        newcnt = jnp.sum((masked == newmax).astype(jnp.int32), axis=1,
                         keepdims=True)
        need_new = cnt <= 0
        val = jnp.where(need_new, newmax, prev)
        cnt = jnp.where(need_new, newcnt, cnt) - 1
        prev = val
        vals.append(val)
    top = jnp.concatenate(vals, axis=1)
    top_ref[:] = top
    mx = top[:, 0:1]
    e = jnp.exp(top - mx)
    attn_ref[:] = e / jnp.sum(e, axis=1, keepdims=True)


def _topk(scores, rb):
    n, m = scores.shape
    return pl.pallas_call(
        _topk_body,
        grid=(n // rb,),
        in_specs=[pl.BlockSpec((rb, m), lambda i: (i, 0))],
        out_specs=[
            pl.BlockSpec((rb, K), lambda i: (i, 0)),
            pl.BlockSpec((rb, K), lambda i: (i, 0)),
        ],
        out_shape=[
            jax.ShapeDtypeStruct((n, K), jnp.float32),
            jax.ShapeDtypeStruct((n, K), jnp.float32),
        ],
    )(scores)


_NROWS_PER_TEC = 32
_M = 8192


def _perm(x, idx):
    return jnp.take_along_axis(x, idx, axis=0, mode="promise_in_bounds")


def _lane_mask(bit):
    t = lax.iota(jnp.int32, 16) & bit
    return (t - 1) >> 31


def _sel(m, a, b):
    ai = lax.bitcast_convert_type(a, jnp.int32)
    bi = lax.bitcast_convert_type(b, jnp.int32)
    return lax.bitcast_convert_type((ai & m) | (bi & ~m), jnp.float32)


def _sort16_desc(x):
    iota = lax.iota(jnp.int32, 16)
    for k in (2, 4, 8, 16):
        dk = _lane_mask(k)
        j = k >> 1
        while j:
            p = _perm(x, iota ^ j)
            tm = ~(_lane_mask(j) ^ dk)
            x = _sel(tm, jnp.maximum(x, p), jnp.minimum(x, p))
            j >>= 1
    return x


def _clean16_desc(x):
    iota = lax.iota(jnp.int32, 16)
    for j in (8, 4, 2, 1):
        p = _perm(x, iota ^ j)
        x = _sel(_lane_mask(j), jnp.maximum(x, p), jnp.minimum(x, p))
    return x


def _rev(x):
    return lax.rev(x, (0,))


def _merge16(a, b):
    rb = _rev(b)
    return (_clean16_desc(jnp.maximum(a, rb)),
            _clean16_desc(jnp.minimum(a, rb)))


def _merge32(p, q):
    r0, r1 = _rev(q[1]), _rev(q[0])
    h0, h1 = jnp.maximum(p[0], r0), jnp.maximum(p[1], r1)
    l0, l1 = jnp.minimum(p[0], r0), jnp.minimum(p[1], r1)
    return (_clean16_desc(jnp.maximum(h0, h1)),
            _clean16_desc(jnp.minimum(h0, h1)),
            _clean16_desc(jnp.maximum(l0, l1)),
            _clean16_desc(jnp.minimum(l0, l1)))


def _group64(v0, v1, v2, v3):
    s = [_sort16_desc(v) for v in (v0, v1, v2, v3)]
    return _merge32(_merge16(s[0], s[1]), _merge16(s[2], s[3]))


def _merge64keep(a, b):
    r = (_rev(b[3]), _rev(b[2]), _rev(b[1]), _rev(b[0]))
    h = [jnp.maximum(a[i], r[i]) for i in range(4)]
    p0, p1 = jnp.maximum(h[0], h[2]), jnp.maximum(h[1], h[3])
    p2, p3 = jnp.minimum(h[0], h[2]), jnp.minimum(h[1], h[3])
    q = (jnp.maximum(p0, p1), jnp.minimum(p0, p1),
         jnp.maximum(p2, p3), jnp.minimum(p2, p3))
    return tuple(_clean16_desc(x) for x in q)


def _bfly(x, op):
    iota = lax.iota(jnp.int32, 16)
    for j in (1, 2, 4, 8):
        x = op(x, _perm(x, iota ^ j))
    return x


def _merge64x16(a, v_sorted):
    h3 = jnp.maximum(a[3], _rev(v_sorted))
    p1, p3 = jnp.maximum(a[1], h3), jnp.minimum(a[1], h3)
    q0, q1 = jnp.maximum(a[0], p1), jnp.minimum(a[0], p1)
    q2, q3 = jnp.maximum(a[2], p3), jnp.minimum(a[2], p3)
    return tuple(_clean16_desc(x) for x in (q0, q1, q2, q3))


def _sc_topk_body(scores_hbm, top_hbm, attn_hbm, row_a, row_b, gmax_v,
                  top_v, attn_v, sem_a, sem_b):
    wid = lax.axis_index("s") * 2 + lax.axis_index("c")
    ninf = jnp.full((16,), NEG_INF, jnp.float32)
    base = wid * _NROWS_PER_TEC
    zero16 = lax.iota(jnp.int32, 16) * 0

    def process(row, row_v):
        def gm_body(gq, _):
            comb = jnp.zeros((16,), jnp.int32)
            for u in range(16):
                g = gq * 16 + u
                v = row_v[pl.ds(g * 64, 16)]
                for k in range(1, 4):
                    v = jnp.maximum(v, row_v[pl.ds(g * 64 + k * 16, 16)])
                sp = _bfly(v, jnp.maximum)
                t = lax.iota(jnp.int32, 16) ^ u
                lane_u = (t - 1) >> 31
                comb = comb | (lax.bitcast_convert_type(sp, jnp.int32) &
                               lane_u)
            gmax_v[pl.ds(gq * 16, 16)] = lax.bitcast_convert_type(
                comb, jnp.float32)
            return 0

        lax.fori_loop(0, 8, gm_body, 0)
        gm = [gmax_v[pl.ds(k * 16, 16)] for k in range(8)]
        pre = _merge64keep(_group64(*gm[:4]), _group64(*gm[4:]))
        t0 = _bfly(pre[3], jnp.minimum)[0]
        for k in range(4):
            top_v[pl.ds(k * 16, 16)] = ninf

        def g_body(g, tfs):
            def hit(tfs_):
                vs = [row_v[pl.ds(g * 64 + k * 16, 16)] for k in range(4)]
                a = tuple(top_v[pl.ds(i * 16, 16)] for i in range(4))
                m = _merge64keep(a, _group64(*vs))
                for i in range(4):
                    top_v[pl.ds(i * 16, 16)] = m[i]
                nt = _bfly(m[3], jnp.minimum)[0]
                return jnp.maximum(tfs_, nt)

            def miss(tfs_):
                return tfs_

            gms = gmax_v[pl.ds(g, 16)][0]
            return lax.cond(gms >= tfs, hit, miss, tfs)

        lax.fori_loop(0, 128, g_body, t0)

        acc = [top_v[pl.ds(k * 16, 16)] for k in range(4)]
        mx = _bfly(acc[0], jnp.maximum)
        es = [jnp.exp(a - mx) for a in acc]
        denom = _bfly(es[0] + es[1] + es[2] + es[3], jnp.add)
        for k in range(4):
            attn_v[pl.ds(k * 16, 16)] = es[k] / denom
        pltpu.sync_copy(top_v, top_hbm.at[row])
        pltpu.sync_copy(attn_v, attn_hbm.at[row])

    pltpu.async_copy(scores_hbm.at[base], row_a, sem_a).wait()

    def pair_body(i, _):
        ra = base + i * 2
        pltpu.async_copy(scores_hbm.at[ra + 1], row_b, sem_b)
        process(ra, row_a)
        pltpu.make_async_copy(scores_hbm.at[ra + 1], row_b, sem_b).wait()

        @pl.when(i < _NROWS_PER_TEC // 2 - 1)
        def _():
            pltpu.async_copy(scores_hbm.at[ra + 2], row_a, sem_a)

        process(ra + 1, row_b)

        @pl.when(i < _NROWS_PER_TEC // 2 - 1)
        def _():
            pltpu.make_async_copy(scores_hbm.at[ra + 2], row_a, sem_a).wait()

        return 0

    lax.fori_loop(0, _NROWS_PER_TEC // 2, pair_body, 0)


def _sc_topk(scores):
    n_rows, m = scores.shape
    mesh = plsc.VectorSubcoreMesh(core_axis_name="c", subcore_axis_name="s")
    f = pl.kernel(
        _sc_topk_body,
        mesh=mesh,
        out_type=[jax.ShapeDtypeStruct((n_rows, K), jnp.float32),
                  jax.ShapeDtypeStruct((n_rows, K), jnp.float32)],
        scratch_types=[pltpu.VMEM((m,), jnp.float32),
                       pltpu.VMEM((m,), jnp.float32),
                       pltpu.VMEM((144,), jnp.float32),
                       pltpu.VMEM((K,), jnp.float32),
                       pltpu.VMEM((K,), jnp.float32),
                       pltpu.SemaphoreType.DMA,
                       pltpu.SemaphoreType.DMA],
    )
    return f(scores)


def _out_body(s_ref, top_ref, q_ref, v_ref, gw_ref, gb_ref, o_ref, w_ref):
    @pl.when(pl.program_id(1) == 0)
    def _build_w():
        s = s_ref[:]
        top = top_ref[:]
        t = top[:, K - 1:K]
        mx = top[:, 0:1]
        denom = jnp.sum(jnp.exp(top - mx), axis=1, keepdims=True)
        gt = s > t
        n_gt = jnp.sum(gt.astype(jnp.int32), axis=1, keepdims=True)
        c_allow = K - n_gt
        eq = s == t
        iota = lax.broadcasted_iota(jnp.int32, s.shape, 1)

        def bs_body(_, carry):
            lo, hi = carry
            mid = (lo + hi) >> 1
            cnt = jnp.sum((eq & (iota <= mid)).astype(jnp.int32), axis=1,
                          keepdims=True)
            pred = cnt >= c_allow
            return jnp.where(pred, lo, mid + 1), jnp.where(pred, mid, hi)

        lo0 = jnp.zeros_like(c_allow)
        hi0 = jnp.full_like(c_allow, s.shape[1] - 1)
        _, bound = lax.fori_loop(0, 13, bs_body, (lo0, hi0))
        sel = gt | (eq & (iota <= bound) & (c_allow > 0))
        e = jnp.exp(s - mx)
        w_ref[:] = (jnp.where(sel, e, 0.0) / denom).astype(jnp.bfloat16)

    w = w_ref[:]
    v = v_ref[:]
    out = lax.dot_general(w, v, (((1,), (0,)), ((), ())),
                          preferred_element_type=jnp.float32,
                          precision=lax.Precision.DEFAULT)
    g = jnp.sum(q_ref[:] * gw_ref[:], axis=1, keepdims=True)
    gate = jax.nn.sigmoid(g + gb_ref[0, 0])
    o_ref[:] = out * gate


def _output(scores, top, qf, values, gate_w, gate_b, rb, hb):
    n, m = scores.shape
    h = values.shape[1]
    grid = (n // rb, h // hb)
    return pl.pallas_call(
        _out_body,
        grid=grid,
        in_specs=[
            pl.BlockSpec((rb, m), lambda i, j: (i, 0)),
            pl.BlockSpec((rb, K), lambda i, j: (i, 0)),
            pl.BlockSpec((rb, qf.shape[1]), lambda i, j: (i, 0)),
            pl.BlockSpec((m, hb), lambda i, j: (0, j)),
            pl.BlockSpec((1, qf.shape[1]), lambda i, j: (0, 0)),
            pl.BlockSpec((1, 1), lambda i, j: (0, 0), memory_space=pltpu.SMEM),
        ],
        out_specs=pl.BlockSpec((rb, hb), lambda i, j: (i, j)),
        out_shape=jax.ShapeDtypeStruct((n, h), jnp.float32),
        scratch_shapes=[pltpu.VMEM((rb, m), jnp.bfloat16)],
    )(scores, top, qf, values, gate_w, gate_b)


def kernel(query, memory_keys, memory_values, gate_w, gate_b, top_k):
    b, q, h = query.shape
    n = b * q
    qf = query.reshape(n, h)
    scores = _scores(qf, memory_keys, rb=128, mb=1024)
    top, attn = _sc_topk(scores)
    gb = jnp.reshape(gate_b, (1, 1)).astype(jnp.float32)
    out = _output(scores, top, qf, memory_values.astype(jnp.bfloat16),
                  gate_w, gb, rb=256, hb=256)
    return out.reshape(b, q, h), attn.reshape(b, q, K)
